# Initial kernel scaffold; baseline (speedup 1.0000x reference)
import jax
import jax.numpy as jnp
from jax.experimental import pallas as pl


def kernel(x, token_ids, mu, gate_up_proj, down_proj, mu_router_w):
    raise NotImplementedError("write your pallas kernel here")


# speedup vs baseline: 7.1751x; 7.1751x over previous
import functools

import jax
import jax.numpy as jnp
from jax.experimental import pallas as pl
from jax.experimental.pallas import tpu as pltpu

H = 768
I = 4096
E = 16
V = 32000
EIS = I // E
T = 512


def _moe_kernel(tid_ref, x_ref, mu_ref, w_ref, gu_ref, dn_ref, out_ref, mask_ref):
    e = pl.program_id(0)

    @pl.when(e == 0)
    def _routing():
        logits = jax.lax.dot_general(
            mu_ref[...], w_ref[...],
            dimension_numbers=(((1,), (1,)), ((), ())),
            preferred_element_type=jnp.float32,
            precision=jax.lax.Precision.HIGHEST,
        )
        tid = tid_ref[...]
        base = jnp.bitwise_and(jnp.clip(tid, 0, V - 1), E - 1)
        iota_e = jax.lax.broadcasted_iota(jnp.int32, (T, E), 1)
        onehot = (base == iota_e).astype(jnp.float32)
        combined = onehot * 10.0 + logits
        m = jnp.max(combined, axis=-1, keepdims=True)
        idx = jnp.min(jnp.where(combined == m, iota_e, E), axis=-1, keepdims=True)
        mask_ref[...] = idx

    h = jax.lax.dot_general(
        x_ref[...], gu_ref[0],
        dimension_numbers=(((1,), (0,)), ((), ())),
        preferred_element_type=jnp.float32,
        precision=jax.lax.Precision.HIGHEST,
    )
    gate = h[:, :EIS]
    up = h[:, EIS:]
    inter = (gate * jax.nn.sigmoid(gate)) * up
    inter = inter * (mask_ref[...] == e).astype(jnp.float32)
    o = jax.lax.dot_general(
        inter, dn_ref[0],
        dimension_numbers=(((1,), (0,)), ((), ())),
        preferred_element_type=jnp.float32,
        precision=jax.lax.Precision.HIGHEST,
    )

    @pl.when(e == 0)
    def _init():
        out_ref[...] = o

    @pl.when(e != 0)
    def _acc():
        out_ref[...] += o


@functools.partial(jax.jit, static_argnames=("interpret",))
def kernel(x, token_ids, mu, gate_up_proj, down_proj, mu_router_w, interpret=False):
    tid2d = token_ids.reshape(T, 1)
    return pl.pallas_call(
        _moe_kernel,
        grid=(E,),
        in_specs=[
            pl.BlockSpec((T, 1), lambda e: (0, 0)),
            pl.BlockSpec((T, H), lambda e: (0, 0)),
            pl.BlockSpec((T, H), lambda e: (0, 0)),
            pl.BlockSpec((E, H), lambda e: (0, 0)),
            pl.BlockSpec((1, H, 2 * EIS), lambda e: (e, 0, 0)),
            pl.BlockSpec((1, EIS, H), lambda e: (e, 0, 0)),
        ],
        out_specs=pl.BlockSpec((T, H), lambda e: (0, 0)),
        out_shape=jax.ShapeDtypeStruct((T, H), jnp.float32),
        scratch_shapes=[pltpu.VMEM((T, 1), jnp.int32)],
        interpret=interpret,
    )(tid2d, x, mu, mu_router_w, gate_up_proj, down_proj)


# speedup vs baseline: 21.0833x; 2.9384x over previous
import functools

import jax
import jax.numpy as jnp
from jax.experimental import pallas as pl
from jax.experimental.pallas import tpu as pltpu

H = 768
I = 4096
E = 16
V = 32000
EIS = I // E
T = 512


def _moe_kernel(tid_ref, x_ref, mu_ref, w_ref, gu_ref, dn_ref, out_ref, mask_ref):
    e = pl.program_id(0)

    @pl.when(e == 0)
    def _routing():
        logits = jax.lax.dot_general(
            mu_ref[...], w_ref[...],
            dimension_numbers=(((1,), (1,)), ((), ())),
            preferred_element_type=jnp.float32,
            precision=jax.lax.Precision.HIGHEST,
        )
        tid = tid_ref[...]
        base = jnp.bitwise_and(jnp.clip(tid, 0, V - 1), E - 1)
        iota_e = jax.lax.broadcasted_iota(jnp.int32, (T, E), 1)
        onehot = (base == iota_e).astype(jnp.float32)
        combined = onehot * 10.0 + logits
        m = jnp.max(combined, axis=-1, keepdims=True)
        idx = jnp.min(jnp.where(combined == m, iota_e, E), axis=-1, keepdims=True)
        mask_ref[...] = idx

    h = jax.lax.dot_general(
        x_ref[...], gu_ref[0],
        dimension_numbers=(((1,), (0,)), ((), ())),
        preferred_element_type=jnp.float32,
        precision=jax.lax.Precision.DEFAULT,
    )
    gate = h[:, :EIS]
    up = h[:, EIS:]
    inter = (gate * jax.nn.sigmoid(gate)) * up
    inter = inter * (mask_ref[...] == e).astype(jnp.float32)
    o = jax.lax.dot_general(
        inter, dn_ref[0],
        dimension_numbers=(((1,), (0,)), ((), ())),
        preferred_element_type=jnp.float32,
        precision=jax.lax.Precision.DEFAULT,
    )

    @pl.when(e == 0)
    def _init():
        out_ref[...] = o

    @pl.when(e != 0)
    def _acc():
        out_ref[...] += o


@functools.partial(jax.jit, static_argnames=("interpret",))
def kernel(x, token_ids, mu, gate_up_proj, down_proj, mu_router_w, interpret=False):
    tid2d = token_ids.reshape(T, 1)
    return pl.pallas_call(
        _moe_kernel,
        grid=(E,),
        in_specs=[
            pl.BlockSpec((T, 1), lambda e: (0, 0)),
            pl.BlockSpec((T, H), lambda e: (0, 0)),
            pl.BlockSpec((T, H), lambda e: (0, 0)),
            pl.BlockSpec((E, H), lambda e: (0, 0)),
            pl.BlockSpec((1, H, 2 * EIS), lambda e: (e, 0, 0)),
            pl.BlockSpec((1, EIS, H), lambda e: (e, 0, 0)),
        ],
        out_specs=pl.BlockSpec((T, H), lambda e: (0, 0)),
        out_shape=jax.ShapeDtypeStruct((T, H), jnp.float32),
        scratch_shapes=[pltpu.VMEM((T, 1), jnp.int32)],
        interpret=interpret,
    )(tid2d, x, mu, mu_router_w, gate_up_proj, down_proj)
